# baseline (speedup 1.0000x reference)
import functools

import jax
import jax.numpy as jnp
from jax import lax
from jax.experimental import pallas as pl
from jax.experimental.pallas import tpu as pltpu
from jax.experimental.pallas import tpu_sc as plsc

BATCH = 4
TOTAL_SEQ = 2048
D_MODEL = 1024
N_POS = 68
ROWS = BATCH * N_POS
ROWS_PER_WORKER = 16
N_CHUNKS = ROWS // ROWS_PER_WORKER


def _flat_positions():
  pos = []
  for b in range(BATCH):
    for j in range(4):
      for k in range(17):
        pos.append(b * TOTAL_SEQ + j * 512 + k * 30)
  return jnp.asarray(pos, dtype=jnp.int32)


def _sc_gather(table, idx):
  mesh = plsc.VectorSubcoreMesh(core_axis_name="c", subcore_axis_name="s")

  @functools.partial(
      pl.kernel,
      mesh=mesh,
      out_type=jax.ShapeDtypeStruct((ROWS, D_MODEL), jnp.float32),
      scratch_types=[
          pltpu.VMEM((ROWS_PER_WORKER,), jnp.int32),
          pltpu.VMEM((ROWS_PER_WORKER, D_MODEL), jnp.float32),
          pltpu.SemaphoreType.DMA,
      ],
  )
  def k(table_hbm, idx_hbm, out_hbm, idx_v, rows_v, sem):
    wid = lax.axis_index("s") * 2 + lax.axis_index("c")

    @pl.when(wid < N_CHUNKS)
    def _():
      base = wid * ROWS_PER_WORKER
      pltpu.sync_copy(idx_hbm.at[pl.ds(base, ROWS_PER_WORKER)], idx_v)
      pltpu.async_copy(table_hbm.at[idx_v], rows_v, sem).wait()
      pltpu.sync_copy(rows_v, out_hbm.at[pl.ds(base, ROWS_PER_WORKER)])

  return k(table, idx)


def kernel(hidden_states):
  table = hidden_states.reshape(BATCH * TOTAL_SEQ, D_MODEL)
  out = _sc_gather(table, _flat_positions())
  return out.reshape(BATCH, N_POS, D_MODEL)
